# baseline (speedup 1.0000x reference)
import functools

import jax
import jax.numpy as jnp
from jax import lax
from jax.experimental import pallas as pl
from jax.experimental.pallas import tpu as pltpu
from jax.experimental.pallas import tpu_sc as plsc

NC = 2
NS = 16
CH = 128
EC = 4096


def _sc_edge_kernel(n_pad, d, ncap, acap):
  cpt = d // (NC * NS)
  mesh = plsc.VectorSubcoreMesh(
      core_axis_name="c", subcore_axis_name="s", num_cores=NC,
      num_subcores=NS)

  @functools.partial(
      pl.kernel,
      mesh=mesh,
      compiler_params=pltpu.CompilerParams(needs_layout_passes=False),
      out_type=jax.ShapeDtypeStruct((d * n_pad,), jnp.float32),
      scratch_types=[
          pltpu.VMEM((cpt * n_pad,), jnp.float32),
          pltpu.VMEM((cpt * n_pad,), jnp.float32),
          pltpu.VMEM((EC,), jnp.int32),
          pltpu.VMEM((EC,), jnp.int32),
          pltpu.VMEM((acap, CH), jnp.int32),
          pltpu.VMEM((CH,), jnp.float32),
          pltpu.SemaphoreType.DMA,
      ],
  )
  def sc_edges(src_h, tgt_h, offs_h, xt_h, zt_h, ones_h, a_ref, aggt_ref,
               xt_v, acc_v, src_v, tgt_v, offs_v, ones_v, sem):
    cid = lax.axis_index("c")
    sid = lax.axis_index("s")
    wid = cid * NS + sid
    cbase = wid * cpt * n_pad

    pltpu.sync_copy(xt_h.at[pl.ds(cbase, cpt * n_pad)], xt_v)
    pltpu.sync_copy(zt_h, acc_v)

    pltpu.sync_copy(ones_h, ones_v)
    pltpu.sync_copy(offs_h.at[pl.ds(wid * acap, acap)], offs_v)

    def fire(g, carry):
      pltpu.async_copy(ones_v, a_ref.at[offs_v.at[g]], sem)
      return carry

    lax.fori_loop(0, acap, fire, 0)

    def chunk(g, carry):
      ebase = g * EC
      pltpu.sync_copy(src_h.at[pl.ds(ebase, EC)], src_v)
      pltpu.sync_copy(tgt_h.at[pl.ds(ebase, EC)], tgt_v)

      def group(q, c2):
        s16 = src_v[pl.ds(q * 16, 16)]
        t16 = tgt_v[pl.ds(q * 16, 16)]
        for c in range(cpt):
          vals = plsc.load_gather(xt_v, [t16 + c * n_pad])
          plsc.addupdate_scatter(acc_v, [s16 + c * n_pad], vals)
        return c2

      lax.fori_loop(0, EC // 16, group, 0)
      return carry

    lax.fori_loop(0, ncap, chunk, 0)

    pltpu.sync_copy(acc_v, aggt_ref.at[pl.ds(cbase, cpt * n_pad)])

    def drain(g, carry):
      pltpu.make_async_copy(ones_v, a_ref.at[offs_v.at[g]], sem).wait()
      return carry

    lax.fori_loop(0, acap, drain, 0)

  return sc_edges


def _tc_fused_kernel(n_pad, d, bm, bn, bj):
  ni, nk, nj = n_pad // bm, n_pad // bn, n_pad // bj

  def body(a1_ref, a2_ref, xk_ref, agg1_ref, w1t_ref, b1_ref, w2t_ref,
           b2_ref, wg1_ref, wg2_ref, bg_ref, out_ref, c_acc, agg2_acc):
    i = pl.program_id(0)
    k = pl.program_id(1)
    j = pl.program_id(2)

    prev = jnp.where(j == 0, 0, c_acc[...])
    c_acc[...] = prev + jnp.dot(a1_ref[...], a2_ref[...],
                                preferred_element_type=jnp.int32)

    @pl.when(j == nj - 1)
    def _():
      rows = i * bm + lax.broadcasted_iota(jnp.int32, (bm, bn), 0)
      cols = k * bn + lax.broadcasted_iota(jnp.int32, (bm, bn), 1)
      thr = jnp.where((c_acc[...] > 0) & (rows != cols),
                      1.0, 0.0).astype(jnp.bfloat16)
      contrib = jnp.dot(thr, xk_ref[...], preferred_element_type=jnp.float32)
      agg2_acc[...] = jnp.where(k == 0, 0.0, agg2_acc[...]) + contrib

      @pl.when(k == nk - 1)
      def _():
        z1 = jnp.dot(agg1_ref[...], w1t_ref[...],
                     preferred_element_type=jnp.float32) + b1_ref[...]
        z2 = jnp.dot(agg2_acc[...], w2t_ref[...],
                     preferred_element_type=jnp.float32) + b2_ref[...]
        gate = jax.nn.sigmoid(
            jnp.dot(z1, wg1_ref[...], preferred_element_type=jnp.float32)
            + jnp.dot(z2, wg2_ref[...], preferred_element_type=jnp.float32)
            + bg_ref[...])
        out_ref[...] = gate * z1 + (1.0 - gate) * z2

  return pl.pallas_call(
      body,
      grid=(ni, nk, nj),
      in_specs=[
          pl.BlockSpec((bm, bj), lambda i, k, j: (i, j)),
          pl.BlockSpec((bj, bn), lambda i, k, j: (j, k)),
          pl.BlockSpec((bn, d), lambda i, k, j: (k, 0)),
          pl.BlockSpec((bm, d), lambda i, k, j: (i, 0)),
          pl.BlockSpec((d, d), lambda i, k, j: (0, 0)),
          pl.BlockSpec((1, d), lambda i, k, j: (0, 0)),
          pl.BlockSpec((d, d), lambda i, k, j: (0, 0)),
          pl.BlockSpec((1, d), lambda i, k, j: (0, 0)),
          pl.BlockSpec((d, d), lambda i, k, j: (0, 0)),
          pl.BlockSpec((d, d), lambda i, k, j: (0, 0)),
          pl.BlockSpec((1, d), lambda i, k, j: (0, 0)),
      ],
      out_specs=pl.BlockSpec((bm, d), lambda i, k, j: (i, 0)),
      out_shape=jax.ShapeDtypeStruct((n_pad, d), jnp.float32),
      scratch_shapes=[
          pltpu.VMEM((bm, bn), jnp.int32),
          pltpu.VMEM((bm, d), jnp.float32),
      ],
      compiler_params=pltpu.CompilerParams(
          dimension_semantics=("parallel", "arbitrary", "arbitrary")),
  )


def kernel(x, edge_index, w1, b1, w2, b2, wg, bg):
  n, d = x.shape
  e = edge_index.shape[1]

  blk = 1024 if n >= 1024 else 128
  n_pad = ((n + blk - 1) // blk) * blk

  src = edge_index[0].astype(jnp.int32)
  tgt = edge_index[1].astype(jnp.int32)
  w = NC * NS
  acap = -(-e // (w * CH))
  e_pad = w * acap * CH
  ncap = e_pad // EC
  pad = e_pad - e
  src_p = jnp.concatenate([src, jnp.full((pad,), n_pad - 1, jnp.int32)])
  tgt_p = jnp.concatenate([tgt, jnp.full((pad,), n_pad - 1, jnp.int32)])
  offs2 = (src_p * n_pad + tgt_p).reshape(w * acap, CH)

  x_pad = jnp.zeros((n_pad, d), jnp.float32).at[:n].set(x)
  xt_flat = x_pad.T.reshape(-1)
  cpt = d // w
  zt_flat = jnp.zeros((cpt * n_pad,), jnp.float32)
  ones_c = jnp.ones((CH,), jnp.float32)

  a_ref = jax.new_ref(jnp.zeros((n_pad * n_pad,), jnp.float32))
  aggt = _sc_edge_kernel(n_pad, d, ncap, acap)(
      src_p, tgt_p, offs2, xt_flat, zt_flat, ones_c, a_ref)
  agg1 = aggt.reshape(d, n_pad).T
  a = a_ref[...].reshape(n_pad, n_pad).astype(jnp.int8)
  x_bf = x_pad.astype(jnp.bfloat16)

  bm = bn = min(2048, n_pad)
  bj = min(1024, n_pad)
  tc = _tc_fused_kernel(n_pad, d, bm, bn, bj)
  out = tc(a, a, x_bf, agg1,
           w1.T, b1.reshape(1, d), w2.T, b2.reshape(1, d),
           wg.T[:d], wg.T[d:], bg.reshape(1, d))
  return out[:n]


# speedup vs baseline: 1.0287x; 1.0287x over previous
import functools

import jax
import jax.numpy as jnp
from jax import lax
from jax.experimental import pallas as pl
from jax.experimental.pallas import tpu as pltpu
from jax.experimental.pallas import tpu_sc as plsc

NC = 2
NS = 16
CH = 128


def _sc_edge_kernel(n_pad, d, ec, ncap, acap):
  cpt = d // (NC * NS)
  mesh = plsc.VectorSubcoreMesh(
      core_axis_name="c", subcore_axis_name="s", num_cores=NC,
      num_subcores=NS)

  @functools.partial(
      pl.kernel,
      mesh=mesh,
      compiler_params=pltpu.CompilerParams(needs_layout_passes=False),
      out_type=jax.ShapeDtypeStruct((d * n_pad,), jnp.float32),
      scratch_types=[
          pltpu.VMEM((cpt * n_pad,), jnp.float32),
          pltpu.VMEM((cpt * n_pad,), jnp.float32),
          pltpu.VMEM((ec,), jnp.int32),
          pltpu.VMEM((ec,), jnp.int32),
          pltpu.VMEM((acap, CH), jnp.int32),
          pltpu.VMEM((CH,), jnp.float32),
          pltpu.SemaphoreType.DMA,
      ],
  )
  def sc_edges(src_h, tgt_h, offs_h, xt_h, zt_h, ones_h, a_ref, aggt_ref,
               xt_v, acc_v, src_v, tgt_v, offs_v, ones_v, sem):
    cid = lax.axis_index("c")
    sid = lax.axis_index("s")
    wid = cid * NS + sid
    cbase = wid * cpt * n_pad

    pltpu.sync_copy(xt_h.at[pl.ds(cbase, cpt * n_pad)], xt_v)
    pltpu.sync_copy(zt_h, acc_v)

    pltpu.sync_copy(ones_h, ones_v)
    pltpu.sync_copy(offs_h.at[pl.ds(wid * acap, acap)], offs_v)

    def fire(g, carry):
      pltpu.async_copy(ones_v, a_ref.at[offs_v.at[g]], sem)
      return carry

    lax.fori_loop(0, acap, fire, 0)

    def chunk(g, carry):
      ebase = g * ec
      pltpu.sync_copy(src_h.at[pl.ds(ebase, ec)], src_v)
      pltpu.sync_copy(tgt_h.at[pl.ds(ebase, ec)], tgt_v)

      def group(q, c2):
        for u in range(4):
          s16 = src_v[pl.ds(q * 64 + u * 16, 16)]
          t16 = tgt_v[pl.ds(q * 64 + u * 16, 16)]
          for c in range(cpt):
            vals = plsc.load_gather(xt_v, [t16 + c * n_pad])
            plsc.addupdate_scatter(acc_v, [s16 + c * n_pad], vals)
        return c2

      lax.fori_loop(0, ec // 64, group, 0)
      return carry

    lax.fori_loop(0, ncap, chunk, 0)

    pltpu.sync_copy(acc_v, aggt_ref.at[pl.ds(cbase, cpt * n_pad)])

    def drain(g, carry):
      pltpu.make_async_copy(ones_v, a_ref.at[offs_v.at[g]], sem).wait()
      return carry

    lax.fori_loop(0, acap, drain, 0)

  return sc_edges


def _tc_fused_kernel(n_pad, d, bm, bn, bj):
  ni, nk, nj = n_pad // bm, n_pad // bn, n_pad // bj

  def body(a1_ref, a2_ref, xk_ref, agg1_ref, w1t_ref, b1_ref, w2t_ref,
           b2_ref, wg1_ref, wg2_ref, bg_ref, out_ref, c_acc, agg2_acc):
    i = pl.program_id(0)
    k = pl.program_id(1)
    j = pl.program_id(2)

    prev = jnp.where(j == 0, 0.0, c_acc[...])
    c_acc[...] = prev + jnp.dot(a1_ref[...], a2_ref[...],
                                preferred_element_type=jnp.float32)

    @pl.when(j == nj - 1)
    def _():
      rows = i * bm + lax.broadcasted_iota(jnp.int32, (bm, bn), 0)
      cols = k * bn + lax.broadcasted_iota(jnp.int32, (bm, bn), 1)
      thr = jnp.where((c_acc[...] > 0.0) & (rows != cols),
                      1.0, 0.0).astype(jnp.bfloat16)
      contrib = jnp.dot(thr, xk_ref[...], preferred_element_type=jnp.float32)
      agg2_acc[...] = jnp.where(k == 0, 0.0, agg2_acc[...]) + contrib

      @pl.when(k == nk - 1)
      def _():
        z1 = jnp.dot(agg1_ref[...], w1t_ref[...],
                     preferred_element_type=jnp.float32) + b1_ref[...]
        z2 = jnp.dot(agg2_acc[...], w2t_ref[...],
                     preferred_element_type=jnp.float32) + b2_ref[...]
        gate = jax.nn.sigmoid(
            jnp.dot(z1, wg1_ref[...], preferred_element_type=jnp.float32)
            + jnp.dot(z2, wg2_ref[...], preferred_element_type=jnp.float32)
            + bg_ref[...])
        out_ref[...] = gate * z1 + (1.0 - gate) * z2

  return pl.pallas_call(
      body,
      grid=(ni, nk, nj),
      in_specs=[
          pl.BlockSpec((bm, bj), lambda i, k, j: (i, j)),
          pl.BlockSpec((bj, bn), lambda i, k, j: (j, k)),
          pl.BlockSpec((bn, d), lambda i, k, j: (k, 0)),
          pl.BlockSpec((bm, d), lambda i, k, j: (i, 0)),
          pl.BlockSpec((d, d), lambda i, k, j: (0, 0)),
          pl.BlockSpec((1, d), lambda i, k, j: (0, 0)),
          pl.BlockSpec((d, d), lambda i, k, j: (0, 0)),
          pl.BlockSpec((1, d), lambda i, k, j: (0, 0)),
          pl.BlockSpec((d, d), lambda i, k, j: (0, 0)),
          pl.BlockSpec((d, d), lambda i, k, j: (0, 0)),
          pl.BlockSpec((1, d), lambda i, k, j: (0, 0)),
      ],
      out_specs=pl.BlockSpec((bm, d), lambda i, k, j: (i, 0)),
      out_shape=jax.ShapeDtypeStruct((n_pad, d), jnp.float32),
      scratch_shapes=[
          pltpu.VMEM((bm, bn), jnp.float32),
          pltpu.VMEM((bm, d), jnp.float32),
      ],
      compiler_params=pltpu.CompilerParams(
          dimension_semantics=("parallel", "arbitrary", "arbitrary")),
  )


def kernel(x, edge_index, w1, b1, w2, b2, wg, bg):
  n, d = x.shape
  e = edge_index.shape[1]

  blk = 1024 if n >= 1024 else 128
  n_pad = ((n + blk - 1) // blk) * blk

  src = edge_index[0].astype(jnp.int32)
  tgt = edge_index[1].astype(jnp.int32)
  w = NC * NS
  acap = -(-e // (w * CH))
  e_pad = w * acap * CH
  ec = 16384 if e_pad % 16384 == 0 else w * CH
  ncap = e_pad // ec
  pad = e_pad - e
  src_p = jnp.concatenate([src, jnp.full((pad,), n_pad - 1, jnp.int32)])
  tgt_p = jnp.concatenate([tgt, jnp.full((pad,), n_pad - 1, jnp.int32)])
  offs2 = (src_p * n_pad + tgt_p).reshape(w * acap, CH)

  x_pad = jnp.zeros((n_pad, d), jnp.float32).at[:n].set(x)
  xt_flat = x_pad.T.reshape(-1)
  cpt = d // w
  zt_flat = jnp.zeros((cpt * n_pad,), jnp.float32)
  ones_c = jnp.ones((CH,), jnp.float32)

  a_ref = jax.new_ref(jnp.zeros((n_pad * n_pad,), jnp.float32))
  aggt = _sc_edge_kernel(n_pad, d, ec, ncap, acap)(
      src_p, tgt_p, offs2, xt_flat, zt_flat, ones_c, a_ref)
  agg1 = aggt.reshape(d, n_pad).T
  a = a_ref[...].reshape(n_pad, n_pad).astype(jnp.bfloat16)
  x_bf = x_pad.astype(jnp.bfloat16)

  bm = bn = min(2048, n_pad)
  bj = min(1024, n_pad)
  tc = _tc_fused_kernel(n_pad, d, bm, bn, bj)
  out = tc(a, a, x_bf, agg1,
           w1.T, b1.reshape(1, d), w2.T, b2.reshape(1, d),
           wg.T[:d], wg.T[d:], bg.reshape(1, d))
  return out[:n]


# speedup vs baseline: 1.0470x; 1.0178x over previous
import functools

import jax
import jax.numpy as jnp
from jax import lax
from jax.experimental import pallas as pl
from jax.experimental.pallas import tpu as pltpu
from jax.experimental.pallas import tpu_sc as plsc

NC = 2
NS = 16
CH = 128


def _sc_edge_kernel(n_pad, d, ec, ncap, acap):
  cpt = d // (NC * NS)
  mesh = plsc.VectorSubcoreMesh(
      core_axis_name="c", subcore_axis_name="s", num_cores=NC,
      num_subcores=NS)

  @functools.partial(
      pl.kernel,
      mesh=mesh,
      compiler_params=pltpu.CompilerParams(needs_layout_passes=False),
      out_type=jax.ShapeDtypeStruct((d * n_pad,), jnp.float32),
      scratch_types=[
          pltpu.VMEM((cpt * n_pad,), jnp.float32),
          pltpu.VMEM((cpt * n_pad,), jnp.float32),
          pltpu.VMEM((ec,), jnp.int32),
          pltpu.VMEM((ec,), jnp.int32),
          pltpu.VMEM((acap, CH), jnp.int32),
          pltpu.VMEM((CH,), jnp.float32),
          pltpu.SemaphoreType.DMA,
      ],
  )
  def sc_edges(src_h, tgt_h, offs_h, xt_h, zt_h, ones_h, a_ref, aggt_ref,
               xt_v, acc_v, src_v, tgt_v, offs_v, ones_v, sem):
    cid = lax.axis_index("c")
    sid = lax.axis_index("s")
    wid = cid * NS + sid
    cbase = wid * cpt * n_pad

    pltpu.sync_copy(xt_h.at[pl.ds(cbase, cpt * n_pad)], xt_v)
    pltpu.sync_copy(zt_h, acc_v)

    pltpu.sync_copy(ones_h, ones_v)
    pltpu.sync_copy(offs_h.at[pl.ds(wid * acap, acap)], offs_v)

    def fire(g, carry):
      pltpu.async_copy(ones_v, a_ref.at[offs_v.at[g]], sem)
      return carry

    lax.fori_loop(0, acap, fire, 0)

    def chunk(g, carry):
      ebase = g * ec
      pltpu.sync_copy(src_h.at[pl.ds(ebase, ec)], src_v)
      pltpu.sync_copy(tgt_h.at[pl.ds(ebase, ec)], tgt_v)

      def group(q, c2):
        for u in range(4):
          s16 = src_v[pl.ds(q * 64 + u * 16, 16)]
          t16 = tgt_v[pl.ds(q * 64 + u * 16, 16)]
          for c in range(cpt):
            vals = plsc.load_gather(xt_v, [t16 + c * n_pad])
            plsc.addupdate_scatter(acc_v, [s16 + c * n_pad], vals)
        return c2

      lax.fori_loop(0, ec // 64, group, 0)
      return carry

    lax.fori_loop(0, ncap, chunk, 0)

    pltpu.sync_copy(acc_v, aggt_ref.at[pl.ds(cbase, cpt * n_pad)])

    def drain(g, carry):
      pltpu.make_async_copy(ones_v, a_ref.at[offs_v.at[g]], sem).wait()
      return carry

    lax.fori_loop(0, acap, drain, 0)

  return sc_edges


def _tc_fused_kernel(n_pad, d, bm, bn, bj):
  ni, nk, nj = n_pad // bm, n_pad // bn, n_pad // bj

  def body(a1_ref, a2_ref, xk_ref, agg1_ref, w1t_ref, b1_ref, w2t_ref,
           b2_ref, wg1_ref, wg2_ref, bg_ref, out_ref, c_acc, agg2_acc):
    i = pl.program_id(0)
    k = pl.program_id(1)
    j = pl.program_id(2)

    prev = jnp.where(j == 0, jnp.bfloat16(0.0), c_acc[...])
    c_acc[...] = prev + jnp.dot(
        a1_ref[...], a2_ref[...],
        preferred_element_type=jnp.float32).astype(jnp.bfloat16)

    @pl.when(j == nj - 1)
    def _():
      rows = i * bm + lax.broadcasted_iota(jnp.int32, (bm, bn), 0)
      cols = k * bn + lax.broadcasted_iota(jnp.int32, (bm, bn), 1)
      thr = jnp.where((c_acc[...].astype(jnp.float32) > 0.0) & (rows != cols),
                      1.0, 0.0).astype(jnp.bfloat16)
      contrib = jnp.dot(thr, xk_ref[...], preferred_element_type=jnp.float32)
      agg2_acc[...] = jnp.where(k == 0, 0.0, agg2_acc[...]) + contrib

      @pl.when(k == nk - 1)
      def _():
        z1 = jnp.dot(agg1_ref[...], w1t_ref[...],
                     preferred_element_type=jnp.float32) + b1_ref[...]
        z2 = jnp.dot(agg2_acc[...], w2t_ref[...],
                     preferred_element_type=jnp.float32) + b2_ref[...]
        gate = jax.nn.sigmoid(
            jnp.dot(z1, wg1_ref[...], preferred_element_type=jnp.float32)
            + jnp.dot(z2, wg2_ref[...], preferred_element_type=jnp.float32)
            + bg_ref[...])
        out_ref[...] = gate * z1 + (1.0 - gate) * z2

  return pl.pallas_call(
      body,
      grid=(ni, nk, nj),
      in_specs=[
          pl.BlockSpec((bm, bj), lambda i, k, j: (i, j)),
          pl.BlockSpec((bj, bn), lambda i, k, j: (j, k)),
          pl.BlockSpec((bn, d), lambda i, k, j: (k, 0)),
          pl.BlockSpec((bm, d), lambda i, k, j: (i, 0)),
          pl.BlockSpec((d, d), lambda i, k, j: (0, 0)),
          pl.BlockSpec((1, d), lambda i, k, j: (0, 0)),
          pl.BlockSpec((d, d), lambda i, k, j: (0, 0)),
          pl.BlockSpec((1, d), lambda i, k, j: (0, 0)),
          pl.BlockSpec((d, d), lambda i, k, j: (0, 0)),
          pl.BlockSpec((d, d), lambda i, k, j: (0, 0)),
          pl.BlockSpec((1, d), lambda i, k, j: (0, 0)),
      ],
      out_specs=pl.BlockSpec((bm, d), lambda i, k, j: (i, 0)),
      out_shape=jax.ShapeDtypeStruct((n_pad, d), jnp.float32),
      scratch_shapes=[
          pltpu.VMEM((bm, bn), jnp.bfloat16),
          pltpu.VMEM((bm, d), jnp.float32),
      ],
      compiler_params=pltpu.CompilerParams(
          dimension_semantics=("parallel", "arbitrary", "arbitrary")),
  )


def kernel(x, edge_index, w1, b1, w2, b2, wg, bg):
  n, d = x.shape
  e = edge_index.shape[1]

  blk = 1024 if n >= 1024 else 128
  n_pad = ((n + blk - 1) // blk) * blk

  src = edge_index[0].astype(jnp.int32)
  tgt = edge_index[1].astype(jnp.int32)
  w = NC * NS
  acap = -(-e // (w * CH))
  e_pad = w * acap * CH
  ec = 16384 if e_pad % 16384 == 0 else w * CH
  ncap = e_pad // ec
  pad = e_pad - e
  src_p = jnp.concatenate([src, jnp.full((pad,), n_pad - 1, jnp.int32)])
  tgt_p = jnp.concatenate([tgt, jnp.full((pad,), n_pad - 1, jnp.int32)])
  offs2 = (src_p * n_pad + tgt_p).reshape(w * acap, CH)

  x_pad = jnp.zeros((n_pad, d), jnp.float32).at[:n].set(x)
  xt_flat = x_pad.T.reshape(-1)
  cpt = d // w
  zt_flat = jnp.zeros((cpt * n_pad,), jnp.float32)
  ones_c = jnp.ones((CH,), jnp.float32)

  a_ref = jax.new_ref(jnp.zeros((n_pad * n_pad,), jnp.float32))
  aggt = _sc_edge_kernel(n_pad, d, ec, ncap, acap)(
      src_p, tgt_p, offs2, xt_flat, zt_flat, ones_c, a_ref)
  agg1 = aggt.reshape(d, n_pad).T
  a = a_ref[...].reshape(n_pad, n_pad).astype(jnp.bfloat16)
  x_bf = x_pad.astype(jnp.bfloat16)

  bm = bn = min(2048, n_pad)
  bj = min(1024, n_pad)
  tc = _tc_fused_kernel(n_pad, d, bm, bn, bj)
  out = tc(a, a, x_bf, agg1,
           w1.T, b1.reshape(1, d), w2.T, b2.reshape(1, d),
           wg.T[:d], wg.T[d:], bg.reshape(1, d))
  return out[:n]
